# baseline (speedup 1.0000x reference)
import jax
import jax.numpy as jnp
from jax import lax
from jax.experimental import pallas as pl
from jax.experimental.pallas import tpu as pltpu
from jax.experimental.pallas import tpu_sc as plsc

_B = 16384
_FIELDS = 26
_DIM = 16
_FIELD_SIZE = 100000
_NC = 2
_NS = 16
_NW = _NC * _NS
_BPW = _B // _NW
_CHUNK = 64
_IDX = _CHUNK * _FIELDS
_NG = _IDX // 128


_GATHER_DNUMS = lax.GatherDimensionNumbers(
    offset_dims=(), collapsed_slice_dims=(0,), start_index_map=(0,)
)


def _lane_perm(v, idx):
    return lax.gather(
        v,
        idx[:, None],
        _GATHER_DNUMS,
        (1,),
        mode=lax.GatherScatterMode.PROMISE_IN_BOUNDS,
    )


def _fm_body(x_hbm, table_hbm, out_hbm, idx_v, rows_v, res_v, sem):
    wid = lax.axis_index("s") * _NC + lax.axis_index("c")
    lanes = lax.broadcasted_iota(jnp.int32, (16,), 0)
    mask0 = lanes == 0

    def chunk(k, _):
        row0 = wid * _BPW + k * _CHUNK
        pltpu.sync_copy(x_hbm.at[pl.ds(row0 * _FIELDS, _IDX)], idx_v)
        for j in range(_IDX // 16):
            f = (j * 16 + lanes) % _FIELDS
            idx_v[pl.ds(j * 16, 16)] = idx_v[pl.ds(j * 16, 16)] + f * _FIELD_SIZE

        copies = [
            pltpu.async_copy(
                table_hbm.at[idx_v.at[pl.ds(g * 128, 128)]],
                rows_v.at[pl.ds(g * 128, 128)],
                sem,
            )
            for g in range(_NG)
        ]
        for c in copies:
            c.wait()

        def row(r, _):
            i0 = r * _FIELDS
            e = rows_v[i0]
            s = e
            q = e * e
            for f in range(1, _FIELDS):
                e = rows_v[i0 + f]
                s = s + e
                q = q + e * e
            t = s * s - q
            for sh in (8, 4, 2, 1):
                t = t + _lane_perm(t, lanes ^ sh)
            plsc.store_scatter(
                res_v,
                [jnp.full((16,), r, jnp.int32)],
                0.5 * t,
                mask=mask0,
            )
            return 0

        lax.fori_loop(0, _CHUNK, row, 0)
        pltpu.sync_copy(res_v, out_hbm.at[pl.ds(row0, _CHUNK)])
        return 0

    lax.fori_loop(0, _BPW // _CHUNK, chunk, 0)


def kernel(x, table):
    xf = x.reshape(-1).astype(jnp.int32)
    mesh = plsc.VectorSubcoreMesh(core_axis_name="c", subcore_axis_name="s")
    out = pl.kernel(
        _fm_body,
        out_type=jax.ShapeDtypeStruct((_B,), jnp.float32),
        mesh=mesh,
        compiler_params=pltpu.CompilerParams(
            needs_layout_passes=False, use_tc_tiling_on_sc=False
        ),
        scratch_types=[
            pltpu.VMEM((_IDX,), jnp.int32),
            pltpu.VMEM((_IDX, _DIM), jnp.float32),
            pltpu.VMEM((_CHUNK,), jnp.float32),
            pltpu.SemaphoreType.DMA,
        ],
    )(xf, table)
    return out.reshape(_B, 1)
